# baseline (speedup 1.0000x reference)
import functools

import jax
import jax.numpy as jnp
from jax import lax
from jax.experimental import pallas as pl
from jax.experimental.pallas import tpu as pltpu
from jax.experimental.pallas import tpu_sc as plsc

_NC, _NS, _L = 2, 16, 16
_NW = _NC * _NS
_C = 128
_PACK = 4


def kernel(user, item, user_table, item_table):
    batch = user.shape[0]
    dim = user_table.shape[1]
    assert batch % (_NW * _C) == 0 and dim == 2 * _L
    bpw = batch // _NW
    nch = bpw // _C
    wide = _PACK * dim

    mesh = plsc.VectorSubcoreMesh(
        core_axis_name="c", subcore_axis_name="s",
        num_cores=_NC, num_subcores=_NS,
    )
    cp = pltpu.CompilerParams(
        needs_layout_passes=False, use_tc_tiling_on_sc=False)

    @functools.partial(
        pl.kernel,
        out_type=jax.ShapeDtypeStruct((batch,), jnp.float32),
        mesh=mesh,
        compiler_params=cp,
        scratch_types=[
            pltpu.VMEM((bpw,), jnp.int32),
            pltpu.VMEM((bpw,), jnp.int32),
            pltpu.VMEM((nch, _C), jnp.int32),
            pltpu.VMEM((nch, _C), jnp.int32),
            pltpu.VMEM((2, _C, wide), jnp.float32),
            pltpu.VMEM((2, _C, wide), jnp.float32),
            pltpu.VMEM((bpw,), jnp.float32),
            pltpu.SemaphoreType.DMA,
            pltpu.SemaphoreType.DMA,
        ],
    )
    def sc_kernel(user_hbm, item_hbm, utab_hbm, itab_hbm, out_hbm,
                  uidx_v, iidx_v, ush_v, ish_v, ubuf, ibuf, out_v,
                  sem_u, sem_i):
        wid = lax.axis_index("s") * _NC + lax.axis_index("c")
        base = wid * bpw
        pltpu.sync_copy(user_hbm.at[pl.ds(base, bpw)], uidx_v)
        pltpu.sync_copy(item_hbm.at[pl.ds(base, bpw)], iidx_v)

        for c in range(nch):
            for j in range(_C // _L):
                s = pl.ds(c * _C + j * _L, _L)
                d = pl.ds(j * _L, _L)
                ush_v.at[c][d] = lax.shift_right_logical(uidx_v[s], 2)
                ish_v.at[c][d] = lax.shift_right_logical(iidx_v[s], 2)

        def issue(c):
            bank = c % 2
            return (
                pltpu.async_copy(utab_hbm.at[ush_v.at[c]], ubuf.at[bank],
                                 sem_u),
                pltpu.async_copy(itab_hbm.at[ish_v.at[c]], ibuf.at[bank],
                                 sem_i),
            )

        lane = lax.iota(jnp.int32, _L)
        inflight = issue(0)
        for c in range(nch):
            nxt = issue(c + 1) if c + 1 < nch else None
            inflight[0].wait()
            inflight[1].wait()
            bank = c % 2
            ur, ir = ubuf.at[bank], ibuf.at[bank]

            @pl.loop(0, _C, step=_L)
            def _(r0):
                g = c * _C + r0
                usub = (uidx_v[pl.ds(g, _L)] & (_PACK - 1)) * dim
                isub = (iidx_v[pl.ds(g, _L)] & (_PACK - 1)) * dim
                rows = r0 + lane
                acc = jnp.zeros((_L,), jnp.float32)
                for d in range(dim):
                    u_d = plsc.load_gather(ur, [rows, usub + d])
                    v_d = plsc.load_gather(ir, [rows, isub + d])
                    acc = acc + u_d * v_d
                out_v[pl.ds(g, _L)] = acc

            inflight = nxt

        pltpu.sync_copy(out_v, out_hbm.at[pl.ds(base, bpw)])

    return sc_kernel(user, item,
                     user_table.reshape(-1, wide),
                     item_table.reshape(-1, wide))


# speedup vs baseline: 1.0015x; 1.0015x over previous
import functools

import jax
import jax.numpy as jnp
from jax import lax
from jax.experimental import pallas as pl
from jax.experimental.pallas import tpu as pltpu
from jax.experimental.pallas import tpu_sc as plsc

_NC, _NS, _L = 2, 16, 16
_NW = _NC * _NS
_C = 128
_PACK = 4


def kernel(user, item, user_table, item_table):
    batch = user.shape[0]
    dim = user_table.shape[1]
    assert batch % (_NW * _C) == 0 and dim == 2 * _L
    bpw = batch // _NW
    nch = bpw // _C
    wide = _PACK * dim

    mesh = plsc.VectorSubcoreMesh(
        core_axis_name="c", subcore_axis_name="s",
        num_cores=_NC, num_subcores=_NS,
    )
    cp = pltpu.CompilerParams(needs_layout_passes=False)

    @functools.partial(
        pl.kernel,
        out_type=jax.ShapeDtypeStruct((batch,), jnp.float32),
        mesh=mesh,
        compiler_params=cp,
        scratch_types=[
            pltpu.VMEM((bpw,), jnp.int32),
            pltpu.VMEM((bpw,), jnp.int32),
            pltpu.VMEM((nch, _C), jnp.int32),
            pltpu.VMEM((nch, _C), jnp.int32),
            pltpu.VMEM((2, _C, wide), jnp.float32),
            pltpu.VMEM((2, _C, wide), jnp.float32),
            pltpu.VMEM((bpw,), jnp.float32),
            pltpu.SemaphoreType.DMA,
            pltpu.SemaphoreType.DMA,
        ],
    )
    def sc_kernel(user_hbm, item_hbm, utab_hbm, itab_hbm, out_hbm,
                  uidx_v, iidx_v, ush_v, ish_v, ubuf, ibuf, out_v,
                  sem_u, sem_i):
        wid = lax.axis_index("s") * _NC + lax.axis_index("c")
        base = wid * bpw
        pltpu.sync_copy(user_hbm.at[pl.ds(base, bpw)], uidx_v)
        pltpu.sync_copy(item_hbm.at[pl.ds(base, bpw)], iidx_v)

        for c in range(nch):
            for j in range(_C // _L):
                s = pl.ds(c * _C + j * _L, _L)
                d = pl.ds(j * _L, _L)
                ush_v.at[c][d] = lax.shift_right_logical(uidx_v[s], 2)
                ish_v.at[c][d] = lax.shift_right_logical(iidx_v[s], 2)

        def issue(c):
            bank = c % 2
            return (
                pltpu.async_copy(utab_hbm.at[ush_v.at[c]], ubuf.at[bank],
                                 sem_u),
                pltpu.async_copy(itab_hbm.at[ish_v.at[c]], ibuf.at[bank],
                                 sem_i),
            )

        lane = lax.iota(jnp.int32, _L)
        inflight = issue(0)
        for c in range(nch):
            nxt = issue(c + 1) if c + 1 < nch else None
            inflight[0].wait()
            inflight[1].wait()
            bank = c % 2
            ur, ir = ubuf.at[bank], ibuf.at[bank]

            @pl.loop(0, _C, step=_L)
            def _(r0):
                g = c * _C + r0
                usub = (uidx_v[pl.ds(g, _L)] & (_PACK - 1)) * dim
                isub = (iidx_v[pl.ds(g, _L)] & (_PACK - 1)) * dim
                rows = r0 + lane
                acc = jnp.zeros((_L,), jnp.float32)
                for d in range(dim):
                    u_d = plsc.load_gather(ur, [rows, usub + d])
                    v_d = plsc.load_gather(ir, [rows, isub + d])
                    acc = acc + u_d * v_d
                out_v[pl.ds(g, _L)] = acc

            inflight = nxt

        pltpu.sync_copy(out_v, out_hbm.at[pl.ds(base, bpw)])

    return sc_kernel(user, item,
                     user_table.reshape(-1, wide),
                     item_table.reshape(-1, wide))
